# baseline (speedup 1.0000x reference)
import functools

import jax
import jax.numpy as jnp
from jax import lax
from jax.experimental import pallas as pl
from jax.experimental.pallas import tpu as pltpu
from jax.experimental.pallas import tpu_sc as plsc

N = 10000
D = 128
E = 320000
K = 128

NC = 2
NS = 16
NW = NC * NS
EPW = E // NW
CH = 64
PCH = 32
NPHASE = 5
NCHUNK = NPHASE * PCH
EPAD = NW * NCHUNK * CH - E
NBUF = 4
OWN = 624
TAIL = N - NS * OWN

SROW = 80
SCOL = 125

_UNPACK_PERM = jnp.array(
    [32 * q + 2 * t + h
     for q in range(4) for h in range(2) for t in range(16)], jnp.int32)


def _sc_agg(nodes, src3, dst3, adj3):
    mesh = plsc.VectorSubcoreMesh(core_axis_name="c", subcore_axis_name="s")

    @functools.partial(
        pl.kernel,
        out_type=jax.ShapeDtypeStruct((NC * N, D), jnp.float32),
        mesh=mesh,
        compiler_params=pltpu.CompilerParams(
            needs_layout_passes=False, use_tc_tiling_on_sc=False),
        scratch_types=dict(
            agg_sh=pltpu.VMEM_SHARED((N, D), jnp.float32),
            ebufs=[pltpu.VMEM((CH, D // 2), jnp.uint32) for _ in range(NBUF)],
            fbufs=[pltpu.VMEM((CH, D), jnp.float32) for _ in range(2)],
            sidx2=pltpu.VMEM((PCH, CH), jnp.int32),
            didx2=pltpu.VMEM((PCH, CH), jnp.int32),
            adj2=pltpu.VMEM((PCH, CH), jnp.float32),
            gsems=[pltpu.SemaphoreType.DMA for _ in range(NBUF)],
            ssems=[pltpu.SemaphoreType.DMA for _ in range(2)],
        ),
    )
    def body(nodes_hbm, src_hbm, dst_hbm, adj_hbm, out_hbm,
             agg_sh, ebufs, fbufs, sidx2, didx2, adj2, gsems, ssems):
        ebufa = fbufs[0]
        ebufb = fbufs[1]
        c = lax.axis_index("c")
        s = lax.axis_index("s")
        wid = s * NC + c

        zeros16 = jnp.zeros((16,), jnp.float32)

        def zrow(r, carry):
            for q in range(D // 16):
                ebufa[r, pl.ds(q * 16, 16)] = zeros16
            return carry

        lax.fori_loop(0, CH, zrow, 0)
        for z in range(OWN // CH):
            pltpu.sync_copy(ebufa, agg_sh.at[pl.ds(s * OWN + z * CH, CH)])
        pltpu.sync_copy(ebufa.at[pl.ds(0, OWN % CH)],
                        agg_sh.at[pl.ds(s * OWN + (OWN // CH) * CH, OWN % CH)])

        @pl.when(s == NS - 1)
        def _zero_tail():
            pltpu.sync_copy(ebufa.at[pl.ds(0, TAIL)],
                            agg_sh.at[pl.ds(NS * OWN, TAIL)])

        plsc.subcore_barrier()

        def gather_start(j, b):
            pltpu.make_async_copy(nodes_hbm.at[sidx2.at[j]],
                                  ebufs[b], gsems[b]).start()

        def gather_wait(j, b):
            pltpu.make_async_copy(nodes_hbm.at[sidx2.at[j]],
                                  ebufs[b], gsems[b]).wait()

        def scatter_start(j, fb):
            pltpu.async_copy(fbufs[fb], agg_sh.at[didx2.at[j]],
                             ssems[fb], add=True)

        def scatter_wait(fb):
            pltpu.make_async_copy(fbufs[fb], agg_sh.at[didx2.at[0]],
                                  ssems[fb]).wait()

        def scale(j, b, fb):
            ebuf = ebufs[b]
            fbuf = fbufs[fb]

            def grp(jj, carry):
                av = adj2[j, pl.ds(jj * 16, 16)]
                for l in range(16):
                    a = av.at[jnp.full((16,), l, jnp.int32)].get(
                        mode="promise_in_bounds")
                    r = jj * 16 + l
                    for q in range(D // 32):
                        vu = ebuf[r, pl.ds(q * 16, 16)]
                        vb = plsc.bitcast(vu, jnp.bfloat16)
                        e0, e1 = plsc.unpack(
                            vb, format=plsc.PackFormat.INTERLEAVED,
                            preferred_element_type=jnp.float32)
                        fbuf[r, pl.ds(q * 32, 16)] = e0 * a
                        fbuf[r, pl.ds(q * 32 + 16, 16)] = e1 * a
                return carry

            lax.fori_loop(0, CH // 16, grp, 0)

        def phase(ph, carry):
            pltpu.sync_copy(src_hbm.at[wid, ph], sidx2)
            pltpu.sync_copy(dst_hbm.at[wid, ph], didx2)
            pltpu.sync_copy(adj_hbm.at[wid, ph], adj2)
            gather_start(0, 0)
            gather_start(1, 1)
            gather_start(2, 2)

            def superblock(i, carry2):
                g0 = NBUF * i
                for b in range(NBUF):
                    g = g0 + b
                    gather_wait(g, b)

                    @pl.when(g >= 2)
                    def _fbuf_wait():
                        scatter_wait(b % 2)

                    scale(g, b, b % 2)
                    scatter_start(g, b % 2)
                    bq = (b + 3) % NBUF

                    @pl.when(g + 3 < PCH)
                    def _prefetch():
                        gather_start(g + 3, bq)

                return carry2

            lax.fori_loop(0, PCH // NBUF, superblock, 0)
            for fb in range(2):
                scatter_wait(fb)
            return carry

        lax.fori_loop(0, NPHASE, phase, 0)
        plsc.subcore_barrier()

        for z in range(OWN // CH):
            r0 = s * OWN + z * CH
            pltpu.sync_copy(agg_sh.at[pl.ds(r0, CH)], ebufa)
            pltpu.sync_copy(ebufa, out_hbm.at[pl.ds(c * N + r0, CH)])
        r0 = s * OWN + (OWN // CH) * CH
        pltpu.sync_copy(agg_sh.at[pl.ds(r0, OWN % CH)],
                        ebufa.at[pl.ds(0, OWN % CH)])
        pltpu.sync_copy(ebufa.at[pl.ds(0, OWN % CH)],
                        out_hbm.at[pl.ds(c * N + r0, OWN % CH)])

        @pl.when(s == NS - 1)
        def _write_tail():
            pltpu.sync_copy(agg_sh.at[pl.ds(NS * OWN, TAIL)],
                            ebufb.at[pl.ds(0, TAIL)])
            pltpu.sync_copy(ebufb.at[pl.ds(0, TAIL)],
                            out_hbm.at[pl.ds(c * N + NS * OWN, TAIL)])

    return body(nodes, src3, dst3, adj3)


def _scores_body(p_ref, n3_ref, o_ref):
    p = p_ref[...]
    pn = p / (jnp.sqrt(jnp.sum(p * p)) + 1e-8)
    for a in range(SROW):
        o_ref[pl.ds(a, 1), :] = lax.dot_general(
            pn, n3_ref[a],
            (((1,), (1,)), ((), ())),
            preferred_element_type=jnp.float32)


def _scores_call(p2, nodes3):
    return pl.pallas_call(
        _scores_body,
        out_shape=jax.ShapeDtypeStruct((SROW, SCOL), jnp.float32),
    )(p2, nodes3)


def _gru_body(sc_ref, nodes_ref, w_ref,
              wz_ref, uz_ref, bz_ref, wr_ref, ur_ref, br_ref,
              wh_ref, uh_ref, bh_ref, out_ref, x_ref):
    lin = (lax.broadcasted_iota(jnp.int32, (SROW, SCOL), 0) * SCOL
           + lax.broadcasted_iota(jnp.int32, (SROW, SCOL), 1))

    def step(t, S):
        m = jnp.max(S)
        amin = jnp.min(jnp.where(S == m, lin, jnp.int32(2**30)))
        row = nodes_ref[pl.ds(amin, 1), :]
        x_ref[pl.ds(t, 1), :] = row * jnp.tanh(m)
        return jnp.where(lin == amin, -jnp.inf, S)

    lax.fori_loop(0, K, step, sc_ref[...])

    X = x_ref[...]
    H = w_ref[...]
    dgT = lambda A, B: lax.dot_general(
        A, B, (((1,), (1,)), ((), ())), preferred_element_type=jnp.float32)
    mm = lambda A, B: jnp.dot(A, B, preferred_element_type=jnp.float32)
    Z = jax.nn.sigmoid(dgT(wz_ref[...], X) + mm(uz_ref[...], H) + bz_ref[...])
    R = jax.nn.sigmoid(dgT(wr_ref[...], X) + mm(ur_ref[...], H) + br_ref[...])
    Ht = jnp.tanh(dgT(wh_ref[...], X) + mm(uh_ref[...], R * H) + bh_ref[...])
    out_ref[...] = (1.0 - Z) * H + Z * Ht


def _gru_call(scores, nodes, weights, Wz, Uz, bz, Wr, Ur, br, Wh, Uh, bh):
    return pl.pallas_call(
        _gru_body,
        out_shape=jax.ShapeDtypeStruct((D, K), jnp.float32),
        scratch_shapes=[pltpu.VMEM((K, D), jnp.float32)],
    )(scores, nodes, weights, Wz, Uz, bz, Wr, Ur, br, Wh, Uh, bh)


_MM_BLK = 1000


def _mm_body(a0_ref, a1_ref, w_ref, o_ref):
    acc = a0_ref[...] + a1_ref[...]
    o_ref[...] = jnp.maximum(
        jnp.dot(acc, w_ref[...], preferred_element_type=jnp.float32), 0.0)


def _mm_call(a0, a1, w):
    return pl.pallas_call(
        _mm_body,
        grid=(N // _MM_BLK,),
        in_specs=[
            pl.BlockSpec((_MM_BLK, D), lambda i: (i, 0)),
            pl.BlockSpec((_MM_BLK, D), lambda i: (i, 0)),
            pl.BlockSpec((D, K), lambda i: (0, 0)),
        ],
        out_specs=pl.BlockSpec((_MM_BLK, K), lambda i: (i, 0)),
        out_shape=jax.ShapeDtypeStruct((N, K), jnp.float32),
    )(a0, a1, w)


def kernel(nodes, adj_values, weights, p, Wz, Uz, bz, Wr, Ur, br, Wh, Uh, bh,
           edge_index):
    nodes_pk = jax.lax.bitcast_convert_type(
        nodes.astype(jnp.bfloat16).reshape(N, D // 2, 2), jnp.uint32)
    spread = (jnp.arange(EPAD, dtype=jnp.int32) * 97) % N
    shape4 = (NW, NPHASE, PCH, CH)
    src3 = jnp.concatenate([edge_index[0], spread]).reshape(shape4)
    dst3 = jnp.concatenate([edge_index[1], spread]).reshape(shape4)
    adj3 = jnp.concatenate(
        [adj_values, jnp.zeros((EPAD,), jnp.float32)]).reshape(shape4)
    agg2 = _sc_agg(nodes_pk, src3, dst3, adj3)
    scores = _scores_call(p.reshape(1, D), nodes.reshape(SROW, SCOL, D))
    weights_new = _gru_call(scores, nodes, weights,
                            Wz, Uz, bz, Wr, Ur, br, Wh, Uh, bh)
    nodes_new = _mm_call(agg2[:N], agg2[N:], weights_new[_UNPACK_PERM])
    return (nodes_new, weights_new)
